# baseline (speedup 1.0000x reference)
import functools

import jax
import jax.numpy as jnp
from jax import lax
from jax.experimental import pallas as pl
from jax.experimental.pallas import tpu as pltpu
from jax.experimental.pallas import tpu_sc as plsc

SEQ_LEN = 200
BATCH = 1024
D_MODEL = 64
B = SEQ_LEN * BATCH
NW = 32
BPW = B // NW
CH = 128
NCH = BPW // CH

_mesh = plsc.VectorSubcoreMesh(core_axis_name="c", subcore_axis_name="s")


@functools.partial(
    pl.kernel,
    mesh=_mesh,
    out_type=jax.ShapeDtypeStruct((B, D_MODEL), jnp.float32),
    scratch_types=[
        pltpu.VMEM((BPW,), jnp.int32),
        pltpu.VMEM((CH, D_MODEL), jnp.float32),
        pltpu.SemaphoreType.DMA,
    ],
    compiler_params=pltpu.CompilerParams(use_tc_tiling_on_sc=False),
)
def _gather(table_hbm, idx_hbm, out_hbm, idx_v, rows_v, sem):
    wid = lax.axis_index("s") * 2 + lax.axis_index("c")
    base = wid * BPW
    pltpu.sync_copy(idx_hbm.at[pl.ds(base, BPW)], idx_v)

    def body(i, carry):
        off = i * CH
        pltpu.async_copy(table_hbm.at[idx_v.at[pl.ds(off, CH)]], rows_v, sem).wait()
        pltpu.sync_copy(rows_v, out_hbm.at[pl.ds(base + off, CH)])
        return carry

    lax.fori_loop(0, NCH, body, 0)


def kernel(input, table):
    flat = input.reshape(B)
    out = _gather(table, flat)
    return out.reshape(SEQ_LEN, BATCH, D_MODEL)


# speedup vs baseline: 1.0471x; 1.0471x over previous
import functools

import jax
import jax.numpy as jnp
from jax import lax
from jax.experimental import pallas as pl
from jax.experimental.pallas import tpu as pltpu
from jax.experimental.pallas import tpu_sc as plsc

SEQ_LEN = 200
BATCH = 1024
D_MODEL = 64
B = SEQ_LEN * BATCH
NW = 32
BPW = B // NW
CH = 128
NCH = BPW // CH
NBUF = 5
NGRP = NCH // NBUF

_mesh = plsc.VectorSubcoreMesh(core_axis_name="c", subcore_axis_name="s")


@functools.partial(
    pl.kernel,
    mesh=_mesh,
    out_type=jax.ShapeDtypeStruct((B, D_MODEL), jnp.float32),
    scratch_types=[
        pltpu.VMEM((BPW,), jnp.int32),
        pltpu.VMEM((NBUF, CH, D_MODEL), jnp.float32),
        pltpu.SemaphoreType.DMA,
    ],
    compiler_params=pltpu.CompilerParams(use_tc_tiling_on_sc=False),
)
def _gather(table_hbm, idx_hbm, out_hbm, idx_v, rows_v, gsem):
    wid = lax.axis_index("s") * 2 + lax.axis_index("c")
    base = wid * BPW
    pltpu.sync_copy(idx_hbm.at[pl.ds(base, BPW)], idx_v)

    def start(chunk, buf):
        pltpu.async_copy(
            table_hbm.at[idx_v.at[pl.ds(chunk * CH, CH)]], rows_v.at[buf], gsem
        )

    def finish(chunk, buf):
        pltpu.make_async_copy(
            table_hbm.at[idx_v.at[pl.ds(chunk * CH, CH)]], rows_v.at[buf], gsem
        ).wait()
        pltpu.sync_copy(rows_v.at[buf], out_hbm.at[pl.ds(base + chunk * CH, CH)])

    for b in range(NBUF):
        start(b, b)

    def body(g, carry):
        for b in range(NBUF):
            i = g * NBUF + b
            finish(i, b)
            start(i + NBUF, b)
        return carry

    lax.fori_loop(0, NGRP - 1, body, 0)

    for b in range(NBUF):
        finish((NGRP - 1) * NBUF + b, b)


def kernel(input, table):
    flat = input.reshape(B)
    out = _gather(table, flat)
    return out.reshape(SEQ_LEN, BATCH, D_MODEL)
